# baseline (speedup 1.0000x reference)
import jax
import jax.numpy as jnp
from jax import lax
from jax.experimental import pallas as pl
from jax.experimental.pallas import tpu as pltpu
from jax.experimental.pallas import tpu_sc as plsc

D_MODEL = 32
D_OUT = 4 * D_MODEL
BATCH = 4096
SEQ = 200
TOTAL = BATCH * SEQ
NC, NS, L = 2, 16, 16
NW = NC * NS
ROWS_PER_W = TOTAL // NW
CHUNK = 256
NCHUNK = ROWS_PER_W // CHUNK
IDX_ROWS = CHUNK // 128
IDX_TOTAL = TOTAL // 128
IDX_PER_W = IDX_TOTAL // NW


def _prep_body(wm_ref, wd_ref, ww_ref, wh_ref, xn_ref,
               t_ref, x0_ref, x1_ref, x2_ref, x3_ref):
    pid = pl.program_id(0)

    @pl.when(pid == 0)
    def _build_table():
        i = lax.broadcasted_iota(jnp.int32, (4096, 1), 0)
        parts = []
        for shift, rows, w_ref in ((9, 8, wm_ref), (6, 8, wd_ref),
                                   (3, 7, ww_ref), (0, 8, wh_ref)):
            sub = (i >> shift) & 7
            acc = jnp.broadcast_to(w_ref[0:1, :], (4096, D_MODEL))
            for k in range(1, rows):
                acc = jnp.where(sub == k, w_ref[k:k + 1, :], acc)
            parts.append(acc)
        t_ref[...] = jnp.concatenate(parts, axis=1)

    xb = xn_ref[...].astype(jnp.bfloat16)
    a = lax.broadcasted_iota(jnp.int32, (512, 128), 0)
    p = lax.broadcasted_iota(jnp.int32, (512, 128), 1)
    for f, o_ref in enumerate((x0_ref, x1_ref, x2_ref, x3_ref)):
        pf = (a == 4 * p + f).astype(jnp.bfloat16)
        o_ref[...] = jnp.dot(
            xb, pf, preferred_element_type=jnp.float32).astype(jnp.int32)


def _gather_body(x0_hbm, x1_hbm, x2_hbm, x3_hbm, t_hbm, out_hbm,
                 t_sh, idx_v, cidx_v, rows_v,
                 sem_i, sem_g0, sem_g1, sem_w0, sem_w1):
    xf_hbm = (x0_hbm, x1_hbm, x2_hbm, x3_hbm)
    sid = lax.axis_index("s")
    wid = sid * NC + lax.axis_index("c")
    idx_base = wid * IDX_PER_W
    out_base = wid * ROWS_PER_W
    sem_g = (sem_g0, sem_g1)
    sem_w = (sem_w0, sem_w1)

    t_rows = 4096 // NS
    pltpu.sync_copy(t_hbm.at[pl.ds(sid * t_rows, t_rows)],
                    t_sh.at[pl.ds(sid * t_rows, t_rows)])
    plsc.subcore_barrier()

    def stage(c, slot):
        for f in range(4):
            pltpu.async_copy(
                xf_hbm[f].at[pl.ds(idx_base + c * IDX_ROWS, IDX_ROWS)],
                idx_v.at[slot, f], sem_i)

    def combine(slot):
        for j in range(IDX_ROWS):
            for p in range(0, 128, L):
                s = pl.ds(p, L)
                cidx_v[slot, j, s] = (
                    (idx_v[slot, 0, j, s] << 9) |
                    (idx_v[slot, 1, j, s] << 6) |
                    (idx_v[slot, 2, j, s] << 3) |
                    idx_v[slot, 3, j, s])

    def fire_gather(slot):
        for j in range(IDX_ROWS):
            pltpu.async_copy(
                t_sh.at[cidx_v.at[slot, j]],
                rows_v.at[slot, pl.ds(j * 128, 128)], sem_g[slot])

    def wait_gather(slot):
        pltpu.make_async_copy(
            out_hbm.at[pl.ds(0, CHUNK)], rows_v.at[slot],
            sem_g[slot]).wait()

    def fire_write(c, slot):
        pltpu.async_copy(
            rows_v.at[slot],
            out_hbm.at[pl.ds(out_base + c * CHUNK, CHUNK)], sem_w[slot])

    def drain_write(slot):
        pltpu.make_async_copy(
            out_hbm.at[pl.ds(0, CHUNK)], rows_v.at[slot],
            sem_w[slot]).wait()

    def wait_stage(slot):
        for f in range(4):
            pltpu.make_async_copy(
                x0_hbm.at[pl.ds(0, IDX_ROWS)], idx_v.at[slot, f],
                sem_i).wait()

    def step(c, b, *, drain, nxt_gather, nxt_stage, last):
        if not last:
            combine(1 - b)
        if nxt_stage:
            stage(c + 2, b)
        if drain:
            drain_write(1 - b)
        if nxt_gather:
            fire_gather(1 - b)
        wait_gather(b)
        fire_write(c, b)
        if nxt_stage:
            wait_stage(b)

    stage(0, 0)
    wait_stage(0)
    combine(0)
    fire_gather(0)
    stage(1, 1)
    wait_stage(1)

    step(0, 0, drain=False, nxt_gather=True, nxt_stage=True, last=False)
    step(1, 1, drain=True, nxt_gather=True, nxt_stage=True, last=False)

    def pair(i, carry):
        c0 = 2 * i
        step(c0, 0, drain=True, nxt_gather=True, nxt_stage=True,
             last=False)
        step(c0 + 1, 1, drain=True, nxt_gather=True, nxt_stage=True,
             last=False)
        return carry

    lax.fori_loop(1, NCHUNK // 2 - 1, pair, 0)

    step(NCHUNK - 2, 0, drain=True, nxt_gather=True, nxt_stage=False,
         last=False)
    step(NCHUNK - 1, 1, drain=True, nxt_gather=False, nxt_stage=False,
         last=True)
    drain_write(1)


PREP_GRID = 4
PREP_ROWS = IDX_TOTAL // PREP_GRID


@jax.jit
def _run(xn, wm, wd, ww, wh):
    xplane = jax.ShapeDtypeStruct((IDX_TOTAL, 128), jnp.int32)
    table, x0, x1, x2, x3 = pl.pallas_call(
        _prep_body,
        grid=(PREP_GRID,),
        in_specs=[
            pl.BlockSpec((13, D_MODEL), lambda i: (0, 0)),
            pl.BlockSpec((32, D_MODEL), lambda i: (0, 0)),
            pl.BlockSpec((7, D_MODEL), lambda i: (0, 0)),
            pl.BlockSpec((24, D_MODEL), lambda i: (0, 0)),
            pl.BlockSpec((PREP_ROWS, 512), lambda i: (i, 0)),
        ],
        out_specs=[
            pl.BlockSpec((4096, D_OUT), lambda i: (0, 0)),
            pl.BlockSpec((PREP_ROWS, 128), lambda i: (i, 0)),
            pl.BlockSpec((PREP_ROWS, 128), lambda i: (i, 0)),
            pl.BlockSpec((PREP_ROWS, 128), lambda i: (i, 0)),
            pl.BlockSpec((PREP_ROWS, 128), lambda i: (i, 0)),
        ],
        out_shape=[jax.ShapeDtypeStruct((4096, D_OUT), jnp.float32),
                   xplane, xplane, xplane, xplane],
    )(wm, wd, ww, wh, xn)

    mesh = plsc.VectorSubcoreMesh(
        core_axis_name="c", subcore_axis_name="s",
        num_cores=NC, num_subcores=NS)
    gather = pl.kernel(
        _gather_body,
        out_type=jax.ShapeDtypeStruct((TOTAL, D_OUT), jnp.float32),
        mesh=mesh,
        scratch_types=[
            pltpu.VMEM_SHARED((4096, D_OUT), jnp.float32),
            pltpu.VMEM((2, 4, IDX_ROWS, 128), jnp.int32),
            pltpu.VMEM((2, IDX_ROWS, 128), jnp.int32),
            pltpu.VMEM((2, CHUNK, D_OUT), jnp.float32),
            pltpu.SemaphoreType.DMA,
            pltpu.SemaphoreType.DMA,
            pltpu.SemaphoreType.DMA,
            pltpu.SemaphoreType.DMA,
            pltpu.SemaphoreType.DMA,
        ],
    )
    return gather(x0, x1, x2, x3, table)


def kernel(x_time, W_month, W_day, W_weekday, W_hour):
    xn = x_time.astype(jnp.int32).reshape(IDX_TOTAL, 512)
    out = _run(xn, W_month, W_day, W_weekday, W_hour)
    return out.reshape(BATCH, SEQ, D_OUT)


# speedup vs baseline: 5.3533x; 5.3533x over previous
import jax
import jax.numpy as jnp
from jax import lax
from jax.experimental import pallas as pl
from jax.experimental.pallas import tpu as pltpu
from jax.experimental.pallas import tpu_sc as plsc

D_MODEL = 32
D_OUT = 4 * D_MODEL
BATCH = 4096
SEQ = 200
TOTAL = BATCH * SEQ
NC, NS, L = 2, 16, 16
NW = NC * NS
ROWS_PER_W = TOTAL // NW
CHUNK = 256
NCHUNK = ROWS_PER_W // CHUNK
IDX_ROWS = CHUNK // 128
IDX_TOTAL = TOTAL // 128
IDX_PER_W = IDX_TOTAL // NW


def _gather_body(xq_hbm, wm_hbm, wd_hbm, ww_hbm, wh_hbm, out_hbm,
                 t_sh, w_v, build_v, idx_v, cidx_v, rows_v,
                 sem_i, sem_g0, sem_g1, sem_w0, sem_w1):
    sid = lax.axis_index("s")
    wid = sid * NC + lax.axis_index("c")
    idx_base = wid * IDX_PER_W
    out_base = wid * ROWS_PER_W
    sem_g = (sem_g0, sem_g1)
    sem_w = (sem_w0, sem_w1)

    pltpu.sync_copy(wm_hbm.at[pl.ds(sid >> 1, 1)], w_v.at[pl.ds(0, 1)])
    pltpu.sync_copy(wd_hbm.at[pl.ds(4 * (sid & 1), 4)],
                    w_v.at[pl.ds(1, 4)])
    pltpu.sync_copy(ww_hbm.at[pl.ds(0, 7)], w_v.at[pl.ds(5, 7)])
    pltpu.sync_copy(wh_hbm.at[pl.ds(0, 8)], w_v.at[pl.ds(12, 8)])

    wrow = [(w_v[r, pl.ds(0, L)], w_v[r, pl.ds(L, L)]) for r in range(20)]
    for k in range(224):
        parts = (wrow[0], wrow[1 + k // 56],
                 wrow[5 + (k // 8) % 7], wrow[12 + (k & 7)])
        for f in range(4):
            build_v[k, pl.ds(2 * f * L, L)] = parts[f][0]
            build_v[k, pl.ds((2 * f + 1) * L, L)] = parts[f][1]
    pltpu.sync_copy(build_v, t_sh.at[pl.ds(sid * 224, 224)])
    plsc.subcore_barrier()

    def stage(c, slot):
        pltpu.async_copy(
            xq_hbm.at[pl.ds(idx_base + c * IDX_ROWS, IDX_ROWS)],
            idx_v.at[slot], sem_i)

    def wait_stage(slot):
        pltpu.make_async_copy(
            xq_hbm.at[pl.ds(0, IDX_ROWS)], idx_v.at[slot], sem_i).wait()

    def combine(slot):
        for j in range(IDX_ROWS):
            for p in range(0, 128, L):
                s = pl.ds(p, L)
                cidx_v[slot, j, s] = (
                    ((((idx_v[slot, j, 0, s] << 3) |
                       idx_v[slot, j, 1, s]) * 7 +
                      idx_v[slot, j, 2, s]) << 3) |
                    idx_v[slot, j, 3, s])

    def fire_gather(slot):
        for j in range(IDX_ROWS):
            pltpu.async_copy(
                t_sh.at[cidx_v.at[slot, j]],
                rows_v.at[slot, pl.ds(j * 128, 128)], sem_g[slot])

    def wait_gather(slot):
        pltpu.make_async_copy(
            out_hbm.at[pl.ds(0, CHUNK)], rows_v.at[slot],
            sem_g[slot]).wait()

    def fire_write(c, slot):
        pltpu.async_copy(
            rows_v.at[slot],
            out_hbm.at[pl.ds(out_base + c * CHUNK, CHUNK)], sem_w[slot])

    def drain_write(slot):
        pltpu.make_async_copy(
            out_hbm.at[pl.ds(0, CHUNK)], rows_v.at[slot],
            sem_w[slot]).wait()

    def step(c, b, *, drain, nxt_gather, nxt_stage, last):
        if not last:
            combine(1 - b)
        if nxt_stage:
            stage(c + 2, b)
        if drain:
            drain_write(1 - b)
        if nxt_gather:
            fire_gather(1 - b)
        wait_gather(b)
        fire_write(c, b)
        if nxt_stage:
            wait_stage(b)

    stage(0, 0)
    wait_stage(0)
    combine(0)
    fire_gather(0)
    stage(1, 1)
    wait_stage(1)

    step(0, 0, drain=False, nxt_gather=True, nxt_stage=True, last=False)
    step(1, 1, drain=True, nxt_gather=True, nxt_stage=True, last=False)

    def pair(i, carry):
        c0 = 2 * i
        step(c0, 0, drain=True, nxt_gather=True, nxt_stage=True,
             last=False)
        step(c0 + 1, 1, drain=True, nxt_gather=True, nxt_stage=True,
             last=False)
        return carry

    lax.fori_loop(1, NCHUNK // 2 - 1, pair, 0)

    step(NCHUNK - 2, 0, drain=True, nxt_gather=True, nxt_stage=False,
         last=False)
    step(NCHUNK - 1, 1, drain=True, nxt_gather=False, nxt_stage=False,
         last=True)
    drain_write(1)


@jax.jit
def _run(xq, wm, wd, ww, wh):
    mesh = plsc.VectorSubcoreMesh(
        core_axis_name="c", subcore_axis_name="s",
        num_cores=NC, num_subcores=NS)
    gather = pl.kernel(
        _gather_body,
        out_type=jax.ShapeDtypeStruct((TOTAL, D_OUT), jnp.float32),
        mesh=mesh,
        scratch_types=[
            pltpu.VMEM_SHARED((3584, D_OUT), jnp.float32),
            pltpu.VMEM((20, D_MODEL), jnp.float32),
            pltpu.VMEM((224, D_OUT), jnp.float32),
            pltpu.VMEM((2, IDX_ROWS, 4, 128), jnp.int32),
            pltpu.VMEM((2, IDX_ROWS, 128), jnp.int32),
            pltpu.VMEM((2, CHUNK, D_OUT), jnp.float32),
            pltpu.SemaphoreType.DMA,
            pltpu.SemaphoreType.DMA,
            pltpu.SemaphoreType.DMA,
            pltpu.SemaphoreType.DMA,
            pltpu.SemaphoreType.DMA,
        ],
    )
    return gather(xq, wm, wd, ww, wh)


def kernel(x_time, W_month, W_day, W_weekday, W_hour):
    xq = x_time.astype(jnp.int32).reshape(IDX_TOTAL, 128, 4).swapaxes(1, 2)
    out = _run(xq, W_month, W_day, W_weekday, W_hour)
    return out.reshape(BATCH, SEQ, D_OUT)


# speedup vs baseline: 5.4689x; 1.0216x over previous
import jax
import jax.numpy as jnp
from jax import lax
from jax.experimental import pallas as pl
from jax.experimental.pallas import tpu as pltpu
from jax.experimental.pallas import tpu_sc as plsc

D_MODEL = 32
D_OUT = 4 * D_MODEL
BATCH = 4096
SEQ = 200
TOTAL = BATCH * SEQ
NC, NS, L = 2, 16, 16
NW = NC * NS
ROWS_PER_W = TOTAL // NW
CHUNK = 256
NCHUNK = ROWS_PER_W // CHUNK
IDX_ROWS = CHUNK // 128
IDX_TOTAL = TOTAL // 128
IDX_PER_W = IDX_TOTAL // NW


def _table_body(wm_ref, wd_ref, ww_ref, wh_ref, t_ref):
    i = lax.broadcasted_iota(jnp.int32, (4096, 1), 0)
    parts = []
    for shift, rows, w_ref in ((9, 8, wm_ref), (6, 8, wd_ref),
                               (3, 7, ww_ref), (0, 8, wh_ref)):
        sub = (i >> shift) & 7
        acc = jnp.broadcast_to(w_ref[0:1, :], (4096, D_MODEL))
        for k in range(1, rows):
            acc = jnp.where(sub == k, w_ref[k:k + 1, :], acc)
        parts.append(acc)
    t_ref[...] = jnp.concatenate(parts, axis=1)


def _gather_body(xq_hbm, t_hbm, out_hbm, t_sh, idx_v, cidx_v, rows_v,
                 sem_i, sem_g0, sem_g1, sem_w0, sem_w1):
    sid = lax.axis_index("s")
    wid = sid * NC + lax.axis_index("c")
    idx_base = wid * IDX_PER_W
    out_base = wid * ROWS_PER_W
    sem_g = (sem_g0, sem_g1)
    sem_w = (sem_w0, sem_w1)

    t_rows = 4096 // NS
    pltpu.sync_copy(t_hbm.at[pl.ds(sid * t_rows, t_rows)],
                    t_sh.at[pl.ds(sid * t_rows, t_rows)])
    plsc.subcore_barrier()

    def stage(c, slot):
        pltpu.async_copy(
            xq_hbm.at[pl.ds(idx_base + c * IDX_ROWS, IDX_ROWS)],
            idx_v.at[slot], sem_i)

    def wait_stage(slot):
        pltpu.make_async_copy(
            xq_hbm.at[pl.ds(0, IDX_ROWS)], idx_v.at[slot], sem_i).wait()

    def combine(slot):
        for j in range(IDX_ROWS):
            for p in range(0, 128, L):
                s = pl.ds(p, L)
                cidx_v[slot, j, s] = (
                    (idx_v[slot, j, 0, s] << 9) |
                    (idx_v[slot, j, 1, s] << 6) |
                    (idx_v[slot, j, 2, s] << 3) |
                    idx_v[slot, j, 3, s])

    def fire_gather(slot):
        for j in range(IDX_ROWS):
            pltpu.async_copy(
                t_sh.at[cidx_v.at[slot, j]],
                rows_v.at[slot, pl.ds(j * 128, 128)], sem_g[slot])

    def wait_gather(slot):
        pltpu.make_async_copy(
            out_hbm.at[pl.ds(0, CHUNK)], rows_v.at[slot],
            sem_g[slot]).wait()

    def fire_write(c, slot):
        pltpu.async_copy(
            rows_v.at[slot],
            out_hbm.at[pl.ds(out_base + c * CHUNK, CHUNK)], sem_w[slot])

    def drain_write(slot):
        pltpu.make_async_copy(
            out_hbm.at[pl.ds(0, CHUNK)], rows_v.at[slot],
            sem_w[slot]).wait()

    def step(c, b, *, drain, nxt_gather, nxt_stage, last):
        if not last:
            combine(1 - b)
        if nxt_stage:
            stage(c + 2, b)
        if drain:
            drain_write(1 - b)
        if nxt_gather:
            fire_gather(1 - b)
        wait_gather(b)
        fire_write(c, b)
        if nxt_stage:
            wait_stage(b)

    stage(0, 0)
    wait_stage(0)
    combine(0)
    fire_gather(0)
    stage(1, 1)
    wait_stage(1)

    step(0, 0, drain=False, nxt_gather=True, nxt_stage=True, last=False)
    step(1, 1, drain=True, nxt_gather=True, nxt_stage=True, last=False)

    def pair(i, carry):
        c0 = 2 * i
        step(c0, 0, drain=True, nxt_gather=True, nxt_stage=True,
             last=False)
        step(c0 + 1, 1, drain=True, nxt_gather=True, nxt_stage=True,
             last=False)
        return carry

    lax.fori_loop(1, NCHUNK // 2 - 1, pair, 0)

    step(NCHUNK - 2, 0, drain=True, nxt_gather=True, nxt_stage=False,
         last=False)
    step(NCHUNK - 1, 1, drain=True, nxt_gather=False, nxt_stage=False,
         last=True)
    drain_write(1)


@jax.jit
def _run(xq, wm, wd, ww, wh):
    table = pl.pallas_call(
        _table_body,
        out_shape=jax.ShapeDtypeStruct((4096, D_OUT), jnp.float32),
    )(wm, wd, ww, wh)

    mesh = plsc.VectorSubcoreMesh(
        core_axis_name="c", subcore_axis_name="s",
        num_cores=NC, num_subcores=NS)
    gather = pl.kernel(
        _gather_body,
        out_type=jax.ShapeDtypeStruct((TOTAL, D_OUT), jnp.float32),
        mesh=mesh,
        scratch_types=[
            pltpu.VMEM_SHARED((4096, D_OUT), jnp.float32),
            pltpu.VMEM((2, IDX_ROWS, 4, 128), jnp.int32),
            pltpu.VMEM((2, IDX_ROWS, 128), jnp.int32),
            pltpu.VMEM((2, CHUNK, D_OUT), jnp.float32),
            pltpu.SemaphoreType.DMA,
            pltpu.SemaphoreType.DMA,
            pltpu.SemaphoreType.DMA,
            pltpu.SemaphoreType.DMA,
            pltpu.SemaphoreType.DMA,
        ],
    )
    return gather(xq, table)


def kernel(x_time, W_month, W_day, W_weekday, W_hour):
    xq = x_time.astype(jnp.int32).reshape(IDX_TOTAL, 128, 4).swapaxes(1, 2)
    out = _run(xq, W_month, W_day, W_weekday, W_hour)
    return out.reshape(BATCH, SEQ, D_OUT)
